# baseline (speedup 1.0000x reference)
import functools

import jax
import jax.numpy as jnp
from jax import lax
from jax.experimental import pallas as pl
from jax.experimental.pallas import tpu as pltpu
from jax.experimental.pallas import tpu_sc as plsc

N = 10000
E = 320000
D = 128
CH = 128
NP = 10240
TILES = 32
K0 = 64
K1 = 96
KMAX = max(K0, K1)
TCH = 16 * (K0 + K1) + 32
EP = TCH * CH
NB = NP // CH
RPT = NP // 16
TB = 128

_mesh = plsc.VectorSubcoreMesh(core_axis_name="c", subcore_axis_name="s")


def _make_sc(with_deg):
    out_type = [jax.ShapeDtypeStruct((2, NP, D), jnp.float32)]
    scratch = [
        pltpu.VMEM((KMAX, CH), jnp.int32),
        pltpu.VMEM((KMAX, CH), jnp.int32),
        pltpu.VMEM((CH, D), jnp.float32),
        pltpu.VMEM((RPT,), jnp.float32),
        pltpu.VMEM((CH,), jnp.float32),
        pltpu.VMEM_SHARED((NP, D), jnp.float32),
        pltpu.SemaphoreType.DMA,
    ]
    if with_deg:
        out_type.append(jax.ShapeDtypeStruct((2, NP), jnp.float32))
        scratch.append(pltpu.VMEM_SHARED((NP,), jnp.float32))

    def body(x_hbm, src_hbm, dst_hbm, p_hbm, *rest):
        if with_deg:
            (deg_hbm, src_v, dst_v, rb, zbuf, ones_v, acc_sh,
             sem, deg_sh) = rest
        else:
            src_v, dst_v, rb, zbuf, ones_v, acc_sh, sem = rest
        c = lax.axis_index("c")
        s = lax.axis_index("s")
        start = jnp.where(c == 0, s * K0, 16 * K0 + s * K1)
        nch = jnp.where(c == 0, K0, K1)

        z16 = jnp.zeros((16,), jnp.float32)

        def zrow(i, carry):
            for k in range(D // 16):
                rb[i, pl.ds(k * 16, 16)] = z16
            return carry

        lax.fori_loop(0, CH, zrow, 0)

        def zz(i, carry):
            zbuf[pl.ds(i * 16, 16)] = z16
            return carry

        lax.fori_loop(0, RPT // 16, zz, 0)
        for k in range(CH // 16):
            ones_v[pl.ds(k * 16, 16)] = jnp.full((16,), 1.0, jnp.float32)

        for b in range(NB // 16):
            r = (s * (NB // 16) + b) * CH
            pltpu.sync_copy(rb, acc_sh.at[pl.ds(r, CH)])
        if with_deg:
            pltpu.sync_copy(zbuf, deg_sh.at[pl.ds(s * RPT, RPT)])
        plsc.subcore_barrier()

        pltpu.sync_copy(src_hbm.at[pl.ds(start, KMAX)], src_v)
        pltpu.sync_copy(dst_hbm.at[pl.ds(start, KMAX)], dst_v)

        def step(j, carry):
            pltpu.async_copy(x_hbm.at[src_v.at[j]], rb, sem).wait()
            pltpu.sync_copy(rb, acc_sh.at[dst_v.at[j]], add=True)
            if with_deg:
                pltpu.sync_copy(ones_v, deg_sh.at[dst_v.at[j]], add=True)
            return carry

        lax.fori_loop(0, nch, step, 0)
        plsc.subcore_barrier()

        for b in range(NB // 16):
            r = (s * (NB // 16) + b) * CH
            pltpu.sync_copy(acc_sh.at[pl.ds(r, CH)], rb)
            pltpu.sync_copy(rb, p_hbm.at[c, pl.ds(r, CH)])
        if with_deg:
            pltpu.sync_copy(deg_sh.at[pl.ds(s * RPT, RPT)], zbuf)
            pltpu.sync_copy(zbuf, deg_hbm.at[c, pl.ds(s * RPT, RPT)])

    return pl.kernel(body, out_type=out_type, mesh=_mesh,
                     scratch_types=scratch)


_sc_layer1 = _make_sc(True)
_sc_layer2 = _make_sc(False)


def _t1_body(h_ref, w1s_ref, x_ref, s1_ref):
    hb = h_ref[...]
    x_ref[...] = hb.T
    s1_ref[...] = lax.dot_general(hb, w1s_ref[...], (((0,), (0,)), ((), ())),
                                  preferred_element_type=jnp.float32)


_t1 = pl.pallas_call(
    _t1_body,
    grid=(NP // TB,),
    in_specs=[pl.BlockSpec((D, TB), lambda i: (0, i)),
              pl.BlockSpec((D, D), lambda i: (0, 0))],
    out_specs=[pl.BlockSpec((TB, D), lambda i: (i, 0)),
               pl.BlockSpec((TB, D), lambda i: (i, 0))],
    out_shape=[jax.ShapeDtypeStruct((NP, D), jnp.float32),
               jax.ShapeDtypeStruct((NP, D), jnp.float32)],
)


def _t2_body(s1_ref, p_ref, deg_ref, w1n_ref, b1_ref, x1_ref):
    p = p_ref[0] + p_ref[1]
    dg = deg_ref[0] + deg_ref[1]
    dinv = 1.0 / jnp.maximum(dg, 1.0)
    agg = p * dinv[:, None]
    acc = (s1_ref[...]
           + lax.dot_general(agg, w1n_ref[...], (((1,), (0,)), ((), ())),
                             preferred_element_type=jnp.float32)
           + b1_ref[...])
    x1_ref[...] = jnp.maximum(acc, 0.0)


_t2 = pl.pallas_call(
    _t2_body,
    grid=(NP // TB,),
    in_specs=[pl.BlockSpec((TB, D), lambda i: (i, 0)),
              pl.BlockSpec((2, TB, D), lambda i: (0, i, 0)),
              pl.BlockSpec((2, TB), lambda i: (0, i)),
              pl.BlockSpec((D, D), lambda i: (0, 0)),
              pl.BlockSpec((1, D), lambda i: (0, 0))],
    out_specs=pl.BlockSpec((TB, D), lambda i: (i, 0)),
    out_shape=jax.ShapeDtypeStruct((NP, D), jnp.float32),
)


def _t3_body(x1_ref, p_ref, deg_ref, w2s_ref, w2n_ref, b2_ref, o_ref):
    p = p_ref[0] + p_ref[1]
    dg = deg_ref[0] + deg_ref[1]
    dinv = 1.0 / jnp.maximum(dg, 1.0)
    agg = p * dinv[:, None]
    acc = (lax.dot_general(w2s_ref[...], x1_ref[...], (((0,), (1,)), ((), ())),
                           preferred_element_type=jnp.float32)
           + lax.dot_general(w2n_ref[...], agg, (((0,), (1,)), ((), ())),
                             preferred_element_type=jnp.float32)
           + b2_ref[...][0][:, None])
    o_ref[...] = acc


_t3 = pl.pallas_call(
    _t3_body,
    grid=(NP // TB,),
    in_specs=[pl.BlockSpec((TB, D), lambda i: (i, 0)),
              pl.BlockSpec((2, TB, D), lambda i: (0, i, 0)),
              pl.BlockSpec((2, TB), lambda i: (0, i)),
              pl.BlockSpec((D, D), lambda i: (0, 0)),
              pl.BlockSpec((D, D), lambda i: (0, 0)),
              pl.BlockSpec((1, D), lambda i: (0, 0))],
    out_specs=pl.BlockSpec((D, TB), lambda i: (0, i)),
    out_shape=jax.ShapeDtypeStruct((D, NP), jnp.float32),
)


def kernel(h, edge_index, W1_self, W1_neigh, b1, W2_self, W2_neigh, b2):
    hp = jnp.pad(h, ((0, 0), (0, NP - N)))
    pad = EP - E
    srcp = jnp.concatenate(
        [edge_index[0], jnp.zeros((pad,), jnp.int32)]).reshape(TCH, CH)
    dstp = jnp.concatenate(
        [edge_index[1], jnp.full((pad,), N, jnp.int32)]).reshape(TCH, CH)

    x, s1 = _t1(hp, W1_self)
    p1, deg = _sc_layer1(x, srcp, dstp)
    x1 = _t2(s1, p1, deg, W1_neigh, b1.reshape(1, D))
    (p2,) = _sc_layer2(x1, srcp, dstp)
    out = _t3(x1, p2, deg, W2_self, W2_neigh, b2.reshape(1, D))
    return out[:, :N]


# speedup vs baseline: 1.0395x; 1.0395x over previous
import functools

import jax
import jax.numpy as jnp
from jax import lax
from jax.experimental import pallas as pl
from jax.experimental.pallas import tpu as pltpu
from jax.experimental.pallas import tpu_sc as plsc

N = 10000
E = 320000
D = 128
CH = 128
NP = 10240
TILES = 32
CPT = 80
G = 16
NG = CPT // G
EP = TILES * CPT * CH
NB = NP // CH
RPT = NP // 16
TB = 128

_mesh = plsc.VectorSubcoreMesh(core_axis_name="c", subcore_axis_name="s")


def _make_sc(with_deg):
    out_type = [jax.ShapeDtypeStruct((2, NP, D), jnp.float32)]
    scratch = [
        pltpu.VMEM((G, CH), jnp.int32),
        pltpu.VMEM((G, CH), jnp.int32),
        pltpu.VMEM((CH, D), jnp.float32),
        pltpu.VMEM((CH, D), jnp.float32),
        pltpu.VMEM((RPT,), jnp.float32),
        pltpu.VMEM((CH,), jnp.float32),
        pltpu.VMEM_SHARED((NP, D), jnp.float32),
        pltpu.SemaphoreType.DMA,
        pltpu.SemaphoreType.DMA,
    ]
    if with_deg:
        out_type.append(jax.ShapeDtypeStruct((2, NP), jnp.float32))
        scratch.append(pltpu.VMEM_SHARED((NP,), jnp.float32))

    def body(x_hbm, src_hbm, dst_hbm, p_hbm, *rest):
        if with_deg:
            (deg_hbm, src_v, dst_v, rb, rb1, zbuf, ones_v, acc_sh,
             sem, sem1, deg_sh) = rest
        else:
            src_v, dst_v, rb, rb1, zbuf, ones_v, acc_sh, sem, sem1 = rest
        c = lax.axis_index("c")
        s = lax.axis_index("s")
        wid = c * 16 + s

        z16 = jnp.zeros((16,), jnp.float32)

        def zrow(i, carry):
            for k in range(D // 16):
                rb[i, pl.ds(k * 16, 16)] = z16
            return carry

        lax.fori_loop(0, CH, zrow, 0)

        def zz(i, carry):
            zbuf[pl.ds(i * 16, 16)] = z16
            return carry

        lax.fori_loop(0, RPT // 16, zz, 0)
        for k in range(CH // 16):
            ones_v[pl.ds(k * 16, 16)] = jnp.full((16,), 1.0, jnp.float32)

        for b in range(NB // 16):
            r = (s * (NB // 16) + b) * CH
            pltpu.sync_copy(rb, acc_sh.at[pl.ds(r, CH)])
        if with_deg:
            pltpu.sync_copy(zbuf, deg_sh.at[pl.ds(s * RPT, RPT)])
        plsc.subcore_barrier()

        def group(g, carry):
            pltpu.sync_copy(src_hbm.at[wid, pl.ds(g * G, G)], src_v)
            pltpu.sync_copy(dst_hbm.at[wid, pl.ds(g * G, G)], dst_v)

            def pair(t, c2):
                d0 = pltpu.async_copy(x_hbm.at[src_v.at[2 * t]], rb, sem)
                d1 = pltpu.async_copy(x_hbm.at[src_v.at[2 * t + 1]], rb1, sem1)
                d0.wait()
                pltpu.sync_copy(rb, acc_sh.at[dst_v.at[2 * t]], add=True)
                if with_deg:
                    pltpu.sync_copy(ones_v, deg_sh.at[dst_v.at[2 * t]],
                                    add=True)
                d1.wait()
                pltpu.sync_copy(rb1, acc_sh.at[dst_v.at[2 * t + 1]], add=True)
                if with_deg:
                    pltpu.sync_copy(ones_v, deg_sh.at[dst_v.at[2 * t + 1]],
                                    add=True)
                return c2

            lax.fori_loop(0, G // 2, pair, 0)
            return carry

        lax.fori_loop(0, NG, group, 0)
        plsc.subcore_barrier()

        for b in range(NB // 16):
            r = (s * (NB // 16) + b) * CH
            pltpu.sync_copy(acc_sh.at[pl.ds(r, CH)], rb)
            pltpu.sync_copy(rb, p_hbm.at[c, pl.ds(r, CH)])
        if with_deg:
            pltpu.sync_copy(deg_sh.at[pl.ds(s * RPT, RPT)], zbuf)
            pltpu.sync_copy(zbuf, deg_hbm.at[c, pl.ds(s * RPT, RPT)])

    return pl.kernel(body, out_type=out_type, mesh=_mesh,
                     scratch_types=scratch)


_sc_layer1 = _make_sc(True)
_sc_layer2 = _make_sc(False)


def _t1_body(h_ref, w1s_ref, x_ref, s1_ref):
    hb = h_ref[...]
    x_ref[...] = hb.T
    s1_ref[...] = lax.dot_general(hb, w1s_ref[...], (((0,), (0,)), ((), ())),
                                  preferred_element_type=jnp.float32)


_t1 = pl.pallas_call(
    _t1_body,
    grid=(NP // TB,),
    in_specs=[pl.BlockSpec((D, TB), lambda i: (0, i)),
              pl.BlockSpec((D, D), lambda i: (0, 0))],
    out_specs=[pl.BlockSpec((TB, D), lambda i: (i, 0)),
               pl.BlockSpec((TB, D), lambda i: (i, 0))],
    out_shape=[jax.ShapeDtypeStruct((NP, D), jnp.float32),
               jax.ShapeDtypeStruct((NP, D), jnp.float32)],
)


def _t2_body(s1_ref, p_ref, deg_ref, w1n_ref, b1_ref, x1_ref):
    p = p_ref[0] + p_ref[1]
    dg = deg_ref[0] + deg_ref[1]
    dinv = 1.0 / jnp.maximum(dg, 1.0)
    agg = p * dinv[:, None]
    acc = (s1_ref[...]
           + lax.dot_general(agg, w1n_ref[...], (((1,), (0,)), ((), ())),
                             preferred_element_type=jnp.float32)
           + b1_ref[...])
    x1_ref[...] = jnp.maximum(acc, 0.0)


_t2 = pl.pallas_call(
    _t2_body,
    grid=(NP // TB,),
    in_specs=[pl.BlockSpec((TB, D), lambda i: (i, 0)),
              pl.BlockSpec((2, TB, D), lambda i: (0, i, 0)),
              pl.BlockSpec((2, TB), lambda i: (0, i)),
              pl.BlockSpec((D, D), lambda i: (0, 0)),
              pl.BlockSpec((1, D), lambda i: (0, 0))],
    out_specs=pl.BlockSpec((TB, D), lambda i: (i, 0)),
    out_shape=jax.ShapeDtypeStruct((NP, D), jnp.float32),
)


def _t3_body(x1_ref, p_ref, deg_ref, w2s_ref, w2n_ref, b2_ref, o_ref):
    p = p_ref[0] + p_ref[1]
    dg = deg_ref[0] + deg_ref[1]
    dinv = 1.0 / jnp.maximum(dg, 1.0)
    agg = p * dinv[:, None]
    acc = (lax.dot_general(w2s_ref[...], x1_ref[...], (((0,), (1,)), ((), ())),
                           preferred_element_type=jnp.float32)
           + lax.dot_general(w2n_ref[...], agg, (((0,), (1,)), ((), ())),
                             preferred_element_type=jnp.float32)
           + b2_ref[...][0][:, None])
    o_ref[...] = acc


_t3 = pl.pallas_call(
    _t3_body,
    grid=(NP // TB,),
    in_specs=[pl.BlockSpec((TB, D), lambda i: (i, 0)),
              pl.BlockSpec((2, TB, D), lambda i: (0, i, 0)),
              pl.BlockSpec((2, TB), lambda i: (0, i)),
              pl.BlockSpec((D, D), lambda i: (0, 0)),
              pl.BlockSpec((D, D), lambda i: (0, 0)),
              pl.BlockSpec((1, D), lambda i: (0, 0))],
    out_specs=pl.BlockSpec((D, TB), lambda i: (0, i)),
    out_shape=jax.ShapeDtypeStruct((D, NP), jnp.float32),
)


def kernel(h, edge_index, W1_self, W1_neigh, b1, W2_self, W2_neigh, b2):
    hp = jnp.pad(h, ((0, 0), (0, NP - N)))
    pad = EP - E
    srcp = jnp.concatenate(
        [edge_index[0], jnp.zeros((pad,), jnp.int32)]).reshape(TILES, CPT, CH)
    dstp = jnp.concatenate(
        [edge_index[1], jnp.full((pad,), N, jnp.int32)]).reshape(TILES, CPT, CH)

    x, s1 = _t1(hp, W1_self)
    p1, deg = _sc_layer1(x, srcp, dstp)
    x1 = _t2(s1, p1, deg, W1_neigh, b1.reshape(1, D))
    (p2,) = _sc_layer2(x1, srcp, dstp)
    out = _t3(x1, p2, deg, W2_self, W2_neigh, b2.reshape(1, D))
    return out[:, :N]


# speedup vs baseline: 1.5024x; 1.4454x over previous
import functools

import jax
import jax.numpy as jnp
from jax import lax
from jax.experimental import pallas as pl
from jax.experimental.pallas import tpu as pltpu
from jax.experimental.pallas import tpu_sc as plsc

N = 10000
E = 320000
D = 128
CH = 128
NP = 10240
TILES = 32
CPT = 79
EP = TILES * CPT * CH
NB = NP // CH
RPT = NP // 16
TB = 128

_mesh = plsc.VectorSubcoreMesh(core_axis_name="c", subcore_axis_name="s")


def _make_sc(with_deg):
    out_type = [jax.ShapeDtypeStruct((2, NP, D), jnp.float32)]
    scratch = [
        pltpu.VMEM((CPT, CH), jnp.int32),
        pltpu.VMEM((CPT, CH), jnp.int32),
        pltpu.VMEM((CH, D), jnp.float32),
        pltpu.VMEM((RPT,), jnp.float32),
        pltpu.VMEM((CH,), jnp.float32),
        pltpu.VMEM_SHARED((NP, D), jnp.float32),
        pltpu.SemaphoreType.DMA,
    ]
    if with_deg:
        out_type.append(jax.ShapeDtypeStruct((2, NP), jnp.float32))
        scratch.append(pltpu.VMEM_SHARED((NP,), jnp.float32))

    def body(x_hbm, src_hbm, dst_hbm, p_hbm, *rest):
        if with_deg:
            deg_hbm, src_v, dst_v, rb, zbuf, ones_v, acc_sh, sem, deg_sh = rest
        else:
            src_v, dst_v, rb, zbuf, ones_v, acc_sh, sem = rest
        c = lax.axis_index("c")
        s = lax.axis_index("s")
        wid = c * 16 + s

        z16 = jnp.zeros((16,), jnp.float32)

        def zrow(i, carry):
            for k in range(D // 16):
                rb[i, pl.ds(k * 16, 16)] = z16
            return carry

        lax.fori_loop(0, CH, zrow, 0)

        def zz(i, carry):
            zbuf[pl.ds(i * 16, 16)] = z16
            return carry

        lax.fori_loop(0, RPT // 16, zz, 0)
        for k in range(CH // 16):
            ones_v[pl.ds(k * 16, 16)] = jnp.full((16,), 1.0, jnp.float32)

        pltpu.sync_copy(src_hbm.at[wid], src_v)
        pltpu.sync_copy(dst_hbm.at[wid], dst_v)

        for b in range(NB // 16):
            r = (s * (NB // 16) + b) * CH
            pltpu.sync_copy(rb, acc_sh.at[pl.ds(r, CH)])
        if with_deg:
            pltpu.sync_copy(zbuf, deg_sh.at[pl.ds(s * RPT, RPT)])
        plsc.subcore_barrier()

        def step(j, carry):
            pltpu.async_copy(x_hbm.at[src_v.at[j]], rb, sem).wait()
            pltpu.sync_copy(rb, acc_sh.at[dst_v.at[j]], add=True)
            if with_deg:
                pltpu.sync_copy(ones_v, deg_sh.at[dst_v.at[j]], add=True)
            return carry

        lax.fori_loop(0, CPT, step, 0)
        plsc.subcore_barrier()

        for b in range(NB // 16):
            r = (s * (NB // 16) + b) * CH
            pltpu.sync_copy(acc_sh.at[pl.ds(r, CH)], rb)
            pltpu.sync_copy(rb, p_hbm.at[c, pl.ds(r, CH)])
        if with_deg:
            pltpu.sync_copy(deg_sh.at[pl.ds(s * RPT, RPT)], zbuf)
            pltpu.sync_copy(zbuf, deg_hbm.at[c, pl.ds(s * RPT, RPT)])

    return pl.kernel(body, out_type=out_type, mesh=_mesh,
                     scratch_types=scratch)


_sc_layer1 = _make_sc(True)
_sc_layer2 = _make_sc(False)


def _t1_body(h_ref, w1s_ref, x_ref, s1_ref):
    hb = h_ref[...]
    x_ref[...] = hb.T
    s1_ref[...] = lax.dot_general(hb, w1s_ref[...], (((0,), (0,)), ((), ())),
                                  preferred_element_type=jnp.float32)


_t1 = pl.pallas_call(
    _t1_body,
    grid=(NP // TB,),
    in_specs=[pl.BlockSpec((D, TB), lambda i: (0, i)),
              pl.BlockSpec((D, D), lambda i: (0, 0))],
    out_specs=[pl.BlockSpec((TB, D), lambda i: (i, 0)),
               pl.BlockSpec((TB, D), lambda i: (i, 0))],
    out_shape=[jax.ShapeDtypeStruct((NP, D), jnp.float32),
               jax.ShapeDtypeStruct((NP, D), jnp.float32)],
)


def _t2_body(s1_ref, p_ref, deg_ref, w1n_ref, b1_ref, x1_ref):
    p = p_ref[0] + p_ref[1]
    dg = deg_ref[0] + deg_ref[1]
    dinv = 1.0 / jnp.maximum(dg, 1.0)
    agg = p * dinv[:, None]
    acc = (s1_ref[...]
           + lax.dot_general(agg, w1n_ref[...], (((1,), (0,)), ((), ())),
                             preferred_element_type=jnp.float32)
           + b1_ref[...])
    x1_ref[...] = jnp.maximum(acc, 0.0)


_t2 = pl.pallas_call(
    _t2_body,
    grid=(NP // TB,),
    in_specs=[pl.BlockSpec((TB, D), lambda i: (i, 0)),
              pl.BlockSpec((2, TB, D), lambda i: (0, i, 0)),
              pl.BlockSpec((2, TB), lambda i: (0, i)),
              pl.BlockSpec((D, D), lambda i: (0, 0)),
              pl.BlockSpec((1, D), lambda i: (0, 0))],
    out_specs=pl.BlockSpec((TB, D), lambda i: (i, 0)),
    out_shape=jax.ShapeDtypeStruct((NP, D), jnp.float32),
)


def _t3_body(x1_ref, p_ref, deg_ref, w2s_ref, w2n_ref, b2_ref, o_ref):
    p = p_ref[0] + p_ref[1]
    dg = deg_ref[0] + deg_ref[1]
    dinv = 1.0 / jnp.maximum(dg, 1.0)
    agg = p * dinv[:, None]
    acc = (lax.dot_general(w2s_ref[...], x1_ref[...], (((0,), (1,)), ((), ())),
                           preferred_element_type=jnp.float32)
           + lax.dot_general(w2n_ref[...], agg, (((0,), (1,)), ((), ())),
                             preferred_element_type=jnp.float32)
           + b2_ref[...][0][:, None])
    o_ref[...] = acc


_t3 = pl.pallas_call(
    _t3_body,
    grid=(NP // TB,),
    in_specs=[pl.BlockSpec((TB, D), lambda i: (i, 0)),
              pl.BlockSpec((2, TB, D), lambda i: (0, i, 0)),
              pl.BlockSpec((2, TB), lambda i: (0, i)),
              pl.BlockSpec((D, D), lambda i: (0, 0)),
              pl.BlockSpec((D, D), lambda i: (0, 0)),
              pl.BlockSpec((1, D), lambda i: (0, 0))],
    out_specs=pl.BlockSpec((D, TB), lambda i: (0, i)),
    out_shape=jax.ShapeDtypeStruct((D, NP), jnp.float32),
)


def kernel(h, edge_index, W1_self, W1_neigh, b1, W2_self, W2_neigh, b2):
    hp = jnp.pad(h, ((0, 0), (0, NP - N)))
    pad = EP - E
    srcp = jnp.concatenate(
        [edge_index[0], jnp.zeros((pad,), jnp.int32)]).reshape(TILES, CPT, CH)
    dstp = jnp.concatenate(
        [edge_index[1], jnp.full((pad,), N, jnp.int32)]).reshape(TILES, CPT, CH)

    x, s1 = _t1(hp, W1_self)
    p1, deg = _sc_layer1(x, srcp, dstp)
    x1 = _t2(s1, p1, deg, W1_neigh, b1.reshape(1, D))
    (p2,) = _sc_layer2(x1, srcp, dstp)
    out = _t3(x1, p2, deg, W2_self, W2_neigh, b2.reshape(1, D))
    return out[:, :N]
